# Initial kernel scaffold; baseline (speedup 1.0000x reference)
import jax
import jax.numpy as jnp
from jax.experimental import pallas as pl


def kernel(x, edge_index, batch, W_msg, W_self, b, att_w, att_b):
    raise NotImplementedError("write your pallas kernel here")


# speedup vs baseline: 6.0899x; 6.0899x over previous
import functools

import jax
import jax.numpy as jnp
from jax import lax
from jax.experimental import pallas as pl
from jax.experimental.pallas import tpu as pltpu
from jax.experimental.pallas import tpu_sc as plsc

_NUM_CORES = 2
_NUM_SUBCORES = 16
_CHUNK = 80


def _dense_pre(x, W_msg, W_self, b_row):
    N, D = x.shape

    def body(x_ref, wm_ref, ws_ref, b_ref, xt_ref, xs_ref):
        xv = x_ref[...]
        xt_ref[...] = jnp.dot(xv, wm_ref[...], preferred_element_type=jnp.float32)
        xs_ref[...] = (
            jnp.dot(xv, ws_ref[...], preferred_element_type=jnp.float32) + b_ref[...]
        )

    return pl.pallas_call(
        body,
        out_shape=[
            jax.ShapeDtypeStruct((N, D), jnp.float32),
            jax.ShapeDtypeStruct((N, D), jnp.float32),
        ],
    )(x, W_msg, W_self, b_row)


def _edge_agg_sc(x_t, src, dst, zeros):
    N, D = x_t.shape
    E = src.shape[0]
    NW = _NUM_CORES * _NUM_SUBCORES
    ew = E // NW
    steps = ew // _CHUNK
    nr = (N // _NUM_SUBCORES) // 8 * 8
    tail = N - nr * _NUM_SUBCORES
    mesh = plsc.VectorSubcoreMesh(core_axis_name="c", subcore_axis_name="s")

    @functools.partial(
        pl.kernel,
        out_type=jax.ShapeDtypeStruct((_NUM_CORES, N, D), jnp.float32),
        mesh=mesh,
        scratch_types=[
            pltpu.VMEM((_CHUNK,), jnp.int32),
            pltpu.VMEM((_CHUNK,), jnp.int32),
            pltpu.VMEM((_CHUNK, D), jnp.float32),
            pltpu.VMEM_SHARED((N, D), jnp.float32),
            pltpu.SemaphoreType.DMA,
        ],
    )
    def k(xt_hbm, src_hbm, dst_hbm, z_hbm, out_hbm, src_v, dst_v, rows_v, agg_sh, sem):
        c = lax.axis_index("c")
        s = lax.axis_index("s")
        wid = c * _NUM_SUBCORES + s
        pltpu.sync_copy(z_hbm.at[pl.ds(s * nr, nr)], agg_sh.at[pl.ds(s * nr, nr)])
        if tail:
            @pl.when(s == _NUM_SUBCORES - 1)
            def _():
                pltpu.sync_copy(
                    z_hbm.at[pl.ds(nr * _NUM_SUBCORES, tail)],
                    agg_sh.at[pl.ds(nr * _NUM_SUBCORES, tail)],
                )
        plsc.subcore_barrier()
        base = wid * ew

        @pl.loop(0, steps)
        def _(i):
            off = base + i * _CHUNK
            pltpu.sync_copy(src_hbm.at[pl.ds(off, _CHUNK)], src_v)
            pltpu.sync_copy(dst_hbm.at[pl.ds(off, _CHUNK)], dst_v)
            pltpu.async_copy(xt_hbm.at[src_v], rows_v, sem).wait()
            pltpu.sync_copy(rows_v, agg_sh.at[dst_v], add=True)

        plsc.subcore_barrier()
        pltpu.sync_copy(
            agg_sh.at[pl.ds(s * nr, nr)], out_hbm.at[c, pl.ds(s * nr, nr)]
        )
        if tail:
            @pl.when(s == _NUM_SUBCORES - 1)
            def _():
                pltpu.sync_copy(
                    agg_sh.at[pl.ds(nr * _NUM_SUBCORES, tail)],
                    out_hbm.at[c, pl.ds(nr * _NUM_SUBCORES, tail)],
                )

    return k(x_t, src, dst, zeros)


def _post(agg2, x_self, att_w_row, batch_row, G):
    N, D = x_self.shape

    def body(agg_ref, xs_ref, aw_ref, bat_ref, out_ref):
        agg = agg_ref[0] + agg_ref[1]
        h = jnp.maximum(agg + xs_ref[...], 0.0)
        logits = lax.dot_general(
            aw_ref[...], h, (((1,), (1,)), ((), ())),
            preferred_element_type=jnp.float32,
        )
        bat = bat_ref[...]
        gid = lax.broadcasted_iota(jnp.int32, (G, N), 0)
        mask = gid == bat
        big_neg = jnp.float32(-1e30)
        seg_max = jnp.max(jnp.where(mask, logits, big_neg), axis=1, keepdims=True)
        maxn = jnp.sum(jnp.where(mask, seg_max, 0.0), axis=0, keepdims=True)
        w = jnp.exp(logits - maxn)
        denom = jnp.sum(jnp.where(mask, w, 0.0), axis=1, keepdims=True)
        denn = jnp.sum(jnp.where(mask, denom, 0.0), axis=0, keepdims=True)
        wn = w / denn
        out_ref[...] = jnp.dot(
            jnp.where(mask, wn, 0.0), h, preferred_element_type=jnp.float32
        )

    return pl.pallas_call(
        body, out_shape=jax.ShapeDtypeStruct((G, D), jnp.float32)
    )(agg2, x_self, att_w_row, batch_row)


def kernel(x, edge_index, batch, W_msg, W_self, b, att_w, att_b):
    N, D = x.shape
    G = 64
    src = edge_index[0]
    dst = edge_index[1]
    x_t, x_self = _dense_pre(x, W_msg, W_self, b.reshape(1, D))
    zeros = jnp.zeros((N, D), jnp.float32)
    agg2 = _edge_agg_sc(x_t, src, dst, zeros)
    out = _post(agg2, x_self, att_w.reshape(1, D), batch.reshape(1, N), G)
    return out


# speedup vs baseline: 11.3603x; 1.8654x over previous
import functools

import jax
import jax.numpy as jnp
from jax import lax
from jax.experimental import pallas as pl
from jax.experimental.pallas import tpu as pltpu
from jax.experimental.pallas import tpu_sc as plsc

_NUM_CORES = 2
_NUM_SUBCORES = 16
_CHUNK = 125


def _dense_pre(x, W_msg, W_self, b_row):
    N, D = x.shape

    def body(x_ref, wm_ref, ws_ref, b_ref, xt_ref, xs_ref):
        xv = x_ref[...]
        xt_ref[...] = jnp.dot(xv, wm_ref[...], preferred_element_type=jnp.float32)
        xs_ref[...] = (
            jnp.dot(xv, ws_ref[...], preferred_element_type=jnp.float32) + b_ref[...]
        )

    return pl.pallas_call(
        body,
        out_shape=[
            jax.ShapeDtypeStruct((N, D), jnp.float32),
            jax.ShapeDtypeStruct((N, D), jnp.float32),
        ],
    )(x, W_msg, W_self, b_row)


def _edge_agg_sc(x_t, src3, dst3, zeros):
    N, D = x_t.shape
    NW, steps, K = src3.shape
    IB = 16
    blocks = steps // IB
    half = IB // 2
    nr = (N // _NUM_SUBCORES) // 8 * 8
    tail = N - nr * _NUM_SUBCORES
    mesh = plsc.VectorSubcoreMesh(core_axis_name="c", subcore_axis_name="s")

    @functools.partial(
        pl.kernel,
        out_type=jax.ShapeDtypeStruct((_NUM_CORES, N, D), jnp.float32),
        mesh=mesh,
        scratch_types=[
            pltpu.VMEM((IB, K), jnp.int32),
            pltpu.VMEM((IB, K), jnp.int32),
            pltpu.VMEM((K, D), jnp.float32),
            pltpu.VMEM((K, D), jnp.float32),
            pltpu.VMEM_SHARED((N, D), jnp.float32),
            pltpu.SemaphoreType.DMA,
            pltpu.SemaphoreType.DMA,
            pltpu.SemaphoreType.DMA,
            pltpu.SemaphoreType.DMA,
        ],
    )
    def k(xt_hbm, src_hbm, dst_hbm, z_hbm, out_hbm,
          src_v, dst_v, rows0, rows1, agg_sh, sg0, sg1, ss0, ss1):
        c = lax.axis_index("c")
        s = lax.axis_index("s")
        wid = c * _NUM_SUBCORES + s
        pltpu.sync_copy(z_hbm.at[pl.ds(s * nr, nr)], agg_sh.at[pl.ds(s * nr, nr)])
        if tail:
            @pl.when(s == _NUM_SUBCORES - 1)
            def _():
                pltpu.sync_copy(
                    z_hbm.at[pl.ds(nr * _NUM_SUBCORES, tail)],
                    agg_sh.at[pl.ds(nr * _NUM_SUBCORES, tail)],
                )
        plsc.subcore_barrier()

        def gather(j, rows, sem):
            return pltpu.async_copy(xt_hbm.at[src_v.at[j]], rows, sem)

        def gather_wait(j, rows, sem):
            pltpu.make_async_copy(xt_hbm.at[src_v.at[j]], rows, sem).wait()

        def scat(j, rows, sem):
            return pltpu.async_copy(rows, agg_sh.at[dst_v.at[j]], sem, add=True)

        def scat_wait(j, rows, sem):
            pltpu.make_async_copy(rows, agg_sh.at[dst_v.at[j]], sem).wait()

        @pl.loop(0, blocks)
        def _(blk):
            pltpu.sync_copy(src_hbm.at[wid, pl.ds(blk * IB, IB)], src_v)
            pltpu.sync_copy(dst_hbm.at[wid, pl.ds(blk * IB, IB)], dst_v)
            gather(0, rows0, sg0)

            @pl.loop(0, half)
            def _(p):
                a = 2 * p
                b = a + 1
                gather_wait(a, rows0, sg0)

                @pl.when(p > 0)
                def _():
                    scat_wait(b, rows1, ss1)

                gather(b, rows1, sg1)
                scat(a, rows0, ss0)
                gather_wait(b, rows1, sg1)
                scat_wait(a, rows0, ss0)

                @pl.when(p < half - 1)
                def _():
                    gather(a + 2, rows0, sg0)

                scat(b, rows1, ss1)

            scat_wait(IB - 1, rows1, ss1)

        plsc.subcore_barrier()
        pltpu.sync_copy(
            agg_sh.at[pl.ds(s * nr, nr)], out_hbm.at[c, pl.ds(s * nr, nr)]
        )
        if tail:
            @pl.when(s == _NUM_SUBCORES - 1)
            def _():
                pltpu.sync_copy(
                    agg_sh.at[pl.ds(nr * _NUM_SUBCORES, tail)],
                    out_hbm.at[c, pl.ds(nr * _NUM_SUBCORES, tail)],
                )

    return k(x_t, src3, dst3, zeros)


def _post(agg2, x_self, att_w_row, batch_row, G):
    N, D = x_self.shape

    def body(agg_ref, xs_ref, aw_ref, bat_ref, out_ref):
        agg = agg_ref[0] + agg_ref[1]
        h = jnp.maximum(agg + xs_ref[...], 0.0)
        logits = lax.dot_general(
            aw_ref[...], h, (((1,), (1,)), ((), ())),
            preferred_element_type=jnp.float32,
        )
        bat = bat_ref[...]
        gid = lax.broadcasted_iota(jnp.int32, (G, N), 0)
        mask = gid == bat
        big_neg = jnp.float32(-1e30)
        seg_max = jnp.max(jnp.where(mask, logits, big_neg), axis=1, keepdims=True)
        maxn = jnp.sum(jnp.where(mask, seg_max, 0.0), axis=0, keepdims=True)
        w = jnp.exp(logits - maxn)
        denom = jnp.sum(jnp.where(mask, w, 0.0), axis=1, keepdims=True)
        denn = jnp.sum(jnp.where(mask, denom, 0.0), axis=0, keepdims=True)
        wn = w / denn
        out_ref[...] = jnp.dot(
            jnp.where(mask, wn, 0.0), h, preferred_element_type=jnp.float32
        )

    return pl.pallas_call(
        body, out_shape=jax.ShapeDtypeStruct((G, D), jnp.float32)
    )(agg2, x_self, att_w_row, batch_row)


def kernel(x, edge_index, batch, W_msg, W_self, b, att_w, att_b):
    N, D = x.shape
    G = 64
    E = edge_index.shape[1]
    NW = _NUM_CORES * _NUM_SUBCORES
    steps = E // (NW * _CHUNK)
    src3 = edge_index[0].reshape(NW, steps, _CHUNK)
    dst3 = edge_index[1].reshape(NW, steps, _CHUNK)
    x_t, x_self = _dense_pre(x, W_msg, W_self, b.reshape(1, D))
    zeros = jnp.zeros((N, D), jnp.float32)
    agg2 = _edge_agg_sc(x_t, src3, dst3, zeros)
    out = _post(agg2, x_self, att_w.reshape(1, D), batch.reshape(1, N), G)
    return out


# speedup vs baseline: 12.5369x; 1.1036x over previous
import functools

import jax
import jax.numpy as jnp
from jax import lax
from jax.experimental import pallas as pl
from jax.experimental.pallas import tpu as pltpu
from jax.experimental.pallas import tpu_sc as plsc

_NUM_CORES = 2
_NUM_SUBCORES = 16
_CHUNK = 125
_IB = 40


def _dense_pre(x, W_msg):
    N, D = x.shape

    def body(x_ref, wm_ref, xt_ref):
        xt_ref[...] = jnp.dot(
            x_ref[...], wm_ref[...], preferred_element_type=jnp.float32
        )

    return pl.pallas_call(
        body, out_shape=jax.ShapeDtypeStruct((N, D), jnp.float32)
    )(x, W_msg)


def _edge_agg_sc(x_t, ei5, zeros):
    N, D = x_t.shape
    _, NW, blocks, IB, K = ei5.shape
    half = IB // 2
    nr = (N // _NUM_SUBCORES) // 8 * 8
    tail = N - nr * _NUM_SUBCORES
    mesh = plsc.VectorSubcoreMesh(core_axis_name="c", subcore_axis_name="s")

    @functools.partial(
        pl.kernel,
        out_type=jax.ShapeDtypeStruct((_NUM_CORES, N, D), jnp.float32),
        mesh=mesh,
        scratch_types=[
            pltpu.VMEM((IB, K), jnp.int32),
            pltpu.VMEM((IB, K), jnp.int32),
            pltpu.VMEM((K, D), jnp.float32),
            pltpu.VMEM((K, D), jnp.float32),
            pltpu.VMEM_SHARED((N, D), jnp.float32),
            pltpu.SemaphoreType.DMA,
            pltpu.SemaphoreType.DMA,
            pltpu.SemaphoreType.DMA,
            pltpu.SemaphoreType.DMA,
        ],
    )
    def k(xt_hbm, ei_hbm, z_hbm, out_hbm,
          src_v, dst_v, rows0, rows1, agg_sh, sg0, sg1, ss0, ss1):
        c = lax.axis_index("c")
        s = lax.axis_index("s")
        wid = c * _NUM_SUBCORES + s
        pltpu.sync_copy(z_hbm.at[pl.ds(0, nr)], agg_sh.at[pl.ds(s * nr, nr)])
        if tail:
            @pl.when(s == _NUM_SUBCORES - 1)
            def _():
                pltpu.sync_copy(
                    z_hbm.at[pl.ds(0, tail)],
                    agg_sh.at[pl.ds(nr * _NUM_SUBCORES, tail)],
                )
        plsc.subcore_barrier()

        def gather(j, rows, sem):
            return pltpu.async_copy(xt_hbm.at[src_v.at[j]], rows, sem)

        def gather_wait(j, rows, sem):
            pltpu.make_async_copy(xt_hbm.at[src_v.at[j]], rows, sem).wait()

        def scat(j, rows, sem):
            return pltpu.async_copy(rows, agg_sh.at[dst_v.at[j]], sem, add=True)

        def scat_wait(j, rows, sem):
            pltpu.make_async_copy(rows, agg_sh.at[dst_v.at[j]], sem).wait()

        for blk in range(blocks):
            pltpu.sync_copy(ei_hbm.at[0, wid, blk], src_v)
            pltpu.sync_copy(ei_hbm.at[1, wid, blk], dst_v)
            gather(0, rows0, sg0)

            @pl.loop(0, half)
            def _(p):
                a = 2 * p
                b = a + 1
                gather_wait(a, rows0, sg0)

                @pl.when(p > 0)
                def _():
                    scat_wait(b, rows1, ss1)

                gather(b, rows1, sg1)
                scat(a, rows0, ss0)
                gather_wait(b, rows1, sg1)
                scat_wait(a, rows0, ss0)

                @pl.when(p < half - 1)
                def _():
                    gather(a + 2, rows0, sg0)

                scat(b, rows1, ss1)

            scat_wait(IB - 1, rows1, ss1)

        plsc.subcore_barrier()
        pltpu.sync_copy(
            agg_sh.at[pl.ds(s * nr, nr)], out_hbm.at[c, pl.ds(s * nr, nr)]
        )
        if tail:
            @pl.when(s == _NUM_SUBCORES - 1)
            def _():
                pltpu.sync_copy(
                    agg_sh.at[pl.ds(nr * _NUM_SUBCORES, tail)],
                    out_hbm.at[c, pl.ds(nr * _NUM_SUBCORES, tail)],
                )

    return k(x_t, ei5, zeros)


def _post(agg2, x, W_self, b_row, att_w_row, batch_row, G):
    N, D = x.shape

    def body(agg_ref, x_ref, ws_ref, b_ref, aw_ref, bat_ref, out_ref):
        x_self = (
            jnp.dot(x_ref[...], ws_ref[...], preferred_element_type=jnp.float32)
            + b_ref[...]
        )
        h = jnp.maximum(agg_ref[0] + agg_ref[1] + x_self, 0.0)
        logits = lax.dot_general(
            aw_ref[...], h, (((1,), (1,)), ((), ())),
            preferred_element_type=jnp.float32,
        )
        bat = bat_ref[...]
        gid = lax.broadcasted_iota(jnp.int32, (G, N), 0)
        mask = gid == bat
        big_neg = jnp.float32(-1e30)
        seg_max = jnp.max(jnp.where(mask, logits, big_neg), axis=1, keepdims=True)
        maxn = jnp.sum(jnp.where(mask, seg_max, 0.0), axis=0, keepdims=True)
        w = jnp.exp(logits - maxn)
        denom = jnp.sum(jnp.where(mask, w, 0.0), axis=1, keepdims=True)
        denn = jnp.sum(jnp.where(mask, denom, 0.0), axis=0, keepdims=True)
        wn = w / denn
        out_ref[...] = jnp.dot(
            jnp.where(mask, wn, 0.0), h, preferred_element_type=jnp.float32
        )

    return pl.pallas_call(
        body, out_shape=jax.ShapeDtypeStruct((G, D), jnp.float32)
    )(agg2, x, W_self, b_row, att_w_row, batch_row)


def kernel(x, edge_index, batch, W_msg, W_self, b, att_w, att_b):
    N, D = x.shape
    G = 64
    E = edge_index.shape[1]
    NW = _NUM_CORES * _NUM_SUBCORES
    steps = E // (NW * _CHUNK)
    blocks = steps // _IB
    ei5 = edge_index.reshape(2, NW, blocks, _IB, _CHUNK)
    x_t = _dense_pre(x, W_msg)
    zeros = jnp.zeros((640, D), jnp.float32)
    agg2 = _edge_agg_sc(x_t, ei5, zeros)
    out = _post(agg2, x, W_self, b.reshape(1, D), att_w.reshape(1, D),
                batch.reshape(1, N), G)
    return out


# speedup vs baseline: 12.7642x; 1.0181x over previous
import functools

import jax
import jax.numpy as jnp
from jax import lax
from jax.experimental import pallas as pl
from jax.experimental.pallas import tpu as pltpu
from jax.experimental.pallas import tpu_sc as plsc

_NUM_CORES = 2
_NUM_SUBCORES = 16
_CHUNK = 125
_IB = 40


def _edge_agg_sc(x_t, ei5, zeros):
    N, D = x_t.shape
    _, NW, blocks, IB, K = ei5.shape
    half = IB // 2
    nr = (N // _NUM_SUBCORES) // 8 * 8
    tail = N - nr * _NUM_SUBCORES
    mesh = plsc.VectorSubcoreMesh(core_axis_name="c", subcore_axis_name="s")

    @functools.partial(
        pl.kernel,
        out_type=jax.ShapeDtypeStruct((_NUM_CORES, N, D), jnp.float32),
        mesh=mesh,
        scratch_types=[
            pltpu.VMEM((IB, K), jnp.int32),
            pltpu.VMEM((IB, K), jnp.int32),
            pltpu.VMEM((K, D), jnp.float32),
            pltpu.VMEM((K, D), jnp.float32),
            pltpu.VMEM_SHARED((N, D), jnp.float32),
            pltpu.SemaphoreType.DMA,
            pltpu.SemaphoreType.DMA,
            pltpu.SemaphoreType.DMA,
            pltpu.SemaphoreType.DMA,
        ],
    )
    def k(xt_hbm, ei_hbm, z_hbm, out_hbm,
          src_v, dst_v, rows0, rows1, agg_sh, sg0, sg1, ss0, ss1):
        c = lax.axis_index("c")
        s = lax.axis_index("s")
        wid = c * _NUM_SUBCORES + s
        pltpu.sync_copy(z_hbm.at[pl.ds(0, nr)], agg_sh.at[pl.ds(s * nr, nr)])
        if tail:
            @pl.when(s == _NUM_SUBCORES - 1)
            def _():
                pltpu.sync_copy(
                    z_hbm.at[pl.ds(0, tail)],
                    agg_sh.at[pl.ds(nr * _NUM_SUBCORES, tail)],
                )
        plsc.subcore_barrier()

        def gather(j, rows, sem):
            return pltpu.async_copy(xt_hbm.at[src_v.at[j]], rows, sem)

        def gather_wait(j, rows, sem):
            pltpu.make_async_copy(xt_hbm.at[src_v.at[j]], rows, sem).wait()

        def scat(j, rows, sem):
            return pltpu.async_copy(rows, agg_sh.at[dst_v.at[j]], sem, add=True)

        def scat_wait(j, rows, sem):
            pltpu.make_async_copy(rows, agg_sh.at[dst_v.at[j]], sem).wait()

        for blk in range(blocks):
            pltpu.sync_copy(ei_hbm.at[0, wid, blk], src_v)
            pltpu.sync_copy(ei_hbm.at[1, wid, blk], dst_v)
            gather(0, rows0, sg0)

            @pl.loop(0, half)
            def _(p):
                a = 2 * p
                b = a + 1
                gather_wait(a, rows0, sg0)

                @pl.when(p > 0)
                def _():
                    scat_wait(b, rows1, ss1)

                gather(b, rows1, sg1)
                scat(a, rows0, ss0)
                gather_wait(b, rows1, sg1)
                scat_wait(a, rows0, ss0)

                @pl.when(p < half - 1)
                def _():
                    gather(a + 2, rows0, sg0)

                scat(b, rows1, ss1)

            scat_wait(IB - 1, rows1, ss1)

        plsc.subcore_barrier()
        pltpu.sync_copy(
            agg_sh.at[pl.ds(s * nr, nr)], out_hbm.at[c, pl.ds(s * nr, nr)]
        )
        if tail:
            @pl.when(s == _NUM_SUBCORES - 1)
            def _():
                pltpu.sync_copy(
                    agg_sh.at[pl.ds(nr * _NUM_SUBCORES, tail)],
                    out_hbm.at[c, pl.ds(nr * _NUM_SUBCORES, tail)],
                )

    return k(x_t, ei5, zeros)


def _post(agg2, x, W_msg, W_self, b_row, att_w_row, batch_row, G):
    N, D = x.shape

    def body(agg_ref, x_ref, wm_ref, ws_ref, b_ref, aw_ref, bat_ref, out_ref):
        aggm = jnp.dot(
            agg_ref[0] + agg_ref[1], wm_ref[...],
            preferred_element_type=jnp.float32,
        )
        x_self = (
            jnp.dot(x_ref[...], ws_ref[...], preferred_element_type=jnp.float32)
            + b_ref[...]
        )
        h = jnp.maximum(aggm + x_self, 0.0)
        logits = lax.dot_general(
            aw_ref[...], h, (((1,), (1,)), ((), ())),
            preferred_element_type=jnp.float32,
        )
        bat = bat_ref[...]
        gid = lax.broadcasted_iota(jnp.int32, (G, N), 0)
        mask = gid == bat
        big_neg = jnp.float32(-1e30)
        seg_max = jnp.max(jnp.where(mask, logits, big_neg), axis=1, keepdims=True)
        maxn = jnp.sum(jnp.where(mask, seg_max, 0.0), axis=0, keepdims=True)
        w = jnp.exp(logits - maxn)
        denom = jnp.sum(jnp.where(mask, w, 0.0), axis=1, keepdims=True)
        denn = jnp.sum(jnp.where(mask, denom, 0.0), axis=0, keepdims=True)
        wn = w / denn
        out_ref[...] = jnp.dot(
            jnp.where(mask, wn, 0.0), h, preferred_element_type=jnp.float32
        )

    return pl.pallas_call(
        body, out_shape=jax.ShapeDtypeStruct((G, D), jnp.float32)
    )(agg2, x, W_msg, W_self, b_row, att_w_row, batch_row)


def kernel(x, edge_index, batch, W_msg, W_self, b, att_w, att_b):
    N, D = x.shape
    G = 64
    E = edge_index.shape[1]
    NW = _NUM_CORES * _NUM_SUBCORES
    steps = E // (NW * _CHUNK)
    blocks = steps // _IB
    ei5 = edge_index.reshape(2, NW, blocks, _IB, _CHUNK)
    zeros = jnp.zeros((640, D), jnp.float32)
    agg2 = _edge_agg_sc(x, ei5, zeros)
    out = _post(agg2, x, W_msg, W_self, b.reshape(1, D), att_w.reshape(1, D),
                batch.reshape(1, N), G)
    return out


# speedup vs baseline: 14.3716x; 1.1259x over previous
import functools

import jax
import jax.numpy as jnp
from jax import lax
from jax.experimental import pallas as pl
from jax.experimental.pallas import tpu as pltpu
from jax.experimental.pallas import tpu_sc as plsc

_NUM_CORES = 2
_NUM_SUBCORES = 16
_K = 128


def _edge_agg_sc(x, edge_index, zeros):
    N, D = x.shape
    E = edge_index.shape[1]
    NW = _NUM_CORES * _NUM_SUBCORES
    chunks = E // _K
    base_cnt = chunks // NW
    extra = chunks - base_cnt * NW
    slots = base_cnt + (1 if extra else 0)
    quads = (slots + 4) // 4
    nr = (N // _NUM_SUBCORES) // 8 * 8
    tail = N - nr * _NUM_SUBCORES
    mesh = plsc.VectorSubcoreMesh(core_axis_name="c", subcore_axis_name="s")

    @functools.partial(
        pl.kernel,
        out_type=jax.ShapeDtypeStruct((_NUM_CORES, N, D), jnp.float32),
        mesh=mesh,
        scratch_types=[
            pltpu.VMEM((4, _K), jnp.int32),
            pltpu.VMEM((4, _K), jnp.int32),
            pltpu.VMEM((_K, D), jnp.float32),
            pltpu.VMEM((_K, D), jnp.float32),
            pltpu.VMEM_SHARED((N, D), jnp.float32),
            pltpu.SemaphoreType.DMA,
            pltpu.SemaphoreType.DMA,
            pltpu.SemaphoreType.DMA,
            pltpu.SemaphoreType.DMA,
            pltpu.SemaphoreType.DMA,
            pltpu.SemaphoreType.DMA,
            pltpu.SemaphoreType.DMA,
            pltpu.SemaphoreType.DMA,
        ],
    )
    def k(x_hbm, ei_hbm, z_hbm, out_hbm,
          srcb, dstb, rows0, rows1, agg_sh,
          sg0, sg1, ss0, ss1, si0, si1, si2, si3):
        c = lax.axis_index("c")
        s = lax.axis_index("s")
        wid = c * _NUM_SUBCORES + s
        start = wid * base_cnt + jnp.minimum(wid, extra)
        cnt = base_cnt + jnp.where(wid < extra, 1, 0)
        rows = (rows0, rows1)
        sg = (sg0, sg1)
        ss = (ss0, ss1)
        si = (si0, si1, si2, si3)

        def idx_copies(slot, r):
            off = (start + slot) * _K
            return (
                pltpu.make_async_copy(ei_hbm.at[0, pl.ds(off, _K)], srcb.at[r], si[r]),
                pltpu.make_async_copy(ei_hbm.at[1, pl.ds(off, _K)], dstb.at[r], si[r]),
            )

        def idx_issue(slot, r):
            for cp in idx_copies(slot, r):
                cp.start()

        def idx_wait(slot, r):
            for cp in idx_copies(slot, r):
                cp.wait()

        def gather_copy(r, par):
            return pltpu.make_async_copy(
                x_hbm.at[srcb.at[r]], rows[par], sg[par]
            )

        def scat_copy(r, par):
            return pltpu.make_async_copy(
                rows[par], agg_sh.at[dstb.at[r]], ss[par]
            )

        pltpu.sync_copy(z_hbm.at[pl.ds(0, nr)], agg_sh.at[pl.ds(s * nr, nr)])
        if tail:
            @pl.when(s == _NUM_SUBCORES - 1)
            def _():
                pltpu.sync_copy(
                    z_hbm.at[pl.ds(0, tail)],
                    agg_sh.at[pl.ds(nr * _NUM_SUBCORES, tail)],
                )
        plsc.subcore_barrier()

        idx_issue(0, 0)
        idx_issue(1, 1)
        idx_issue(2, 2)
        idx_wait(0, 0)
        gather_copy(0, 0).start()

        @pl.loop(0, quads)
        def _(q):
            for j in range(4):
                i = 4 * q + j
                r_i = j
                r_n = (j + 1) % 4
                r_f = (j + 3) % 4
                p_i = j % 2
                p_n = (j + 1) % 2

                @pl.when(jnp.logical_and(i >= 1, i - 1 < cnt))
                def _():
                    scat_copy((j + 3) % 4, p_n).wait()

                @pl.when(i + 1 < cnt)
                def _():
                    idx_wait(i + 1, r_n)
                    gather_copy(r_n, p_n).start()

                @pl.when(i < cnt)
                def _():
                    gather_copy(r_i, p_i).wait()
                    scat_copy(r_i, p_i).start(add=True)

                @pl.when(i + 3 < cnt)
                def _():
                    idx_issue(i + 3, r_f)

        plsc.subcore_barrier()
        pltpu.sync_copy(
            agg_sh.at[pl.ds(s * nr, nr)], out_hbm.at[c, pl.ds(s * nr, nr)]
        )
        if tail:
            @pl.when(s == _NUM_SUBCORES - 1)
            def _():
                pltpu.sync_copy(
                    agg_sh.at[pl.ds(nr * _NUM_SUBCORES, tail)],
                    out_hbm.at[c, pl.ds(nr * _NUM_SUBCORES, tail)],
                )

    return k(x, edge_index, zeros)


def _post(agg2, x, W_msg, W_self, b_row, att_w_row, batch_row, G):
    N, D = x.shape

    def body(agg_ref, x_ref, wm_ref, ws_ref, b_ref, aw_ref, bat_ref, out_ref):
        aggm = jnp.dot(
            agg_ref[0] + agg_ref[1], wm_ref[...],
            preferred_element_type=jnp.float32,
        )
        x_self = (
            jnp.dot(x_ref[...], ws_ref[...], preferred_element_type=jnp.float32)
            + b_ref[...]
        )
        h = jnp.maximum(aggm + x_self, 0.0)
        logits = lax.dot_general(
            aw_ref[...], h, (((1,), (1,)), ((), ())),
            preferred_element_type=jnp.float32,
        )
        bat = bat_ref[...]
        gid = lax.broadcasted_iota(jnp.int32, (G, N), 0)
        mask = gid == bat
        big_neg = jnp.float32(-1e30)
        seg_max = jnp.max(jnp.where(mask, logits, big_neg), axis=1, keepdims=True)
        maxn = jnp.sum(jnp.where(mask, seg_max, 0.0), axis=0, keepdims=True)
        w = jnp.exp(logits - maxn)
        denom = jnp.sum(jnp.where(mask, w, 0.0), axis=1, keepdims=True)
        denn = jnp.sum(jnp.where(mask, denom, 0.0), axis=0, keepdims=True)
        wn = w / denn
        out_ref[...] = jnp.dot(
            jnp.where(mask, wn, 0.0), h, preferred_element_type=jnp.float32
        )

    return pl.pallas_call(
        body, out_shape=jax.ShapeDtypeStruct((G, D), jnp.float32)
    )(agg2, x, W_msg, W_self, b_row, att_w_row, batch_row)


def kernel(x, edge_index, batch, W_msg, W_self, b, att_w, att_b):
    N, D = x.shape
    G = 64
    zeros = jnp.zeros((640, D), jnp.float32)
    agg2 = _edge_agg_sc(x, edge_index, zeros)
    out = _post(agg2, x, W_msg, W_self, b.reshape(1, D), att_w.reshape(1, D),
                batch.reshape(1, N), G)
    return out


# speedup vs baseline: 16.4331x; 1.1434x over previous
import functools

import jax
import jax.numpy as jnp
from jax import lax
from jax.experimental import pallas as pl
from jax.experimental.pallas import tpu as pltpu
from jax.experimental.pallas import tpu_sc as plsc

_NUM_CORES = 2
_NUM_SUBCORES = 16
_K = 128


def _edge_agg_sc(x, edge_index, zeros):
    N, D = x.shape
    E = edge_index.shape[1]
    NW = _NUM_CORES * _NUM_SUBCORES
    chunks = E // _K
    base_cnt = chunks // NW
    extra = chunks - base_cnt * NW
    slots = base_cnt + (1 if extra else 0)
    unroll = 12
    iters = (slots + 2 + unroll) // unroll
    nr = (N // _NUM_SUBCORES) // 8 * 8
    tail = N - nr * _NUM_SUBCORES
    mesh = plsc.VectorSubcoreMesh(core_axis_name="c", subcore_axis_name="s")

    @functools.partial(
        pl.kernel,
        out_type=jax.ShapeDtypeStruct((_NUM_CORES, N, D), jnp.float32),
        mesh=mesh,
        scratch_types=[
            pltpu.VMEM((4, _K), jnp.int32),
            pltpu.VMEM((4, _K), jnp.int32),
            pltpu.VMEM((_K, D), jnp.float32),
            pltpu.VMEM((_K, D), jnp.float32),
            pltpu.VMEM((_K, D), jnp.float32),
            pltpu.VMEM_SHARED((N, D), jnp.float32),
        ] + [pltpu.SemaphoreType.DMA] * 14,
    )
    def k(x_hbm, ei_hbm, z_hbm, out_hbm,
          srcb, dstb, rows0, rows1, rows2, agg_sh,
          sg0, sg1, sg2, ss0, ss1, ss2,
          sa0, sa1, sa2, sa3, sd0, sd1, sd2, sd3):
        c = lax.axis_index("c")
        s = lax.axis_index("s")
        wid = c * _NUM_SUBCORES + s
        start = wid * base_cnt + jnp.minimum(wid, extra)
        cnt = base_cnt + jnp.where(wid < extra, 1, 0)
        rows = (rows0, rows1, rows2)
        sg = (sg0, sg1, sg2)
        ss = (ss0, ss1, ss2)
        sa = (sa0, sa1, sa2, sa3)
        sd = (sd0, sd1, sd2, sd3)

        def src_copy(slot, r):
            off = (start + slot) * _K
            return pltpu.make_async_copy(
                ei_hbm.at[0, pl.ds(off, _K)], srcb.at[r], sa[r]
            )

        def dst_copy(slot, r):
            off = (start + slot) * _K
            return pltpu.make_async_copy(
                ei_hbm.at[1, pl.ds(off, _K)], dstb.at[r], sd[r]
            )

        def gather_copy(r, par):
            return pltpu.make_async_copy(
                x_hbm.at[srcb.at[r]], rows[par], sg[par]
            )

        def scat_copy(r, par):
            return pltpu.make_async_copy(
                rows[par], agg_sh.at[dstb.at[r]], ss[par]
            )

        pltpu.sync_copy(z_hbm.at[pl.ds(0, nr)], agg_sh.at[pl.ds(s * nr, nr)])
        if tail:
            @pl.when(s == _NUM_SUBCORES - 1)
            def _():
                pltpu.sync_copy(
                    z_hbm.at[pl.ds(0, tail)],
                    agg_sh.at[pl.ds(nr * _NUM_SUBCORES, tail)],
                )
        plsc.subcore_barrier()

        src_copy(0, 0).start()
        src_copy(1, 1).start()
        src_copy(2, 2).start()
        dst_copy(0, 0).start()
        dst_copy(1, 1).start()
        src_copy(0, 0).wait()
        gather_copy(0, 0).start()

        @pl.loop(0, iters)
        def _(q):
            for j in range(unroll):
                i = unroll * q + j

                @pl.when(jnp.logical_and(i >= 2, i - 2 < cnt))
                def _():
                    scat_copy((j + 2) % 4, (j + 1) % 3).wait()

                @pl.when(i + 1 < cnt)
                def _():
                    src_copy(i + 1, (j + 1) % 4).wait()
                    gather_copy((j + 1) % 4, (j + 1) % 3).start()

                @pl.when(i < cnt)
                def _():
                    gather_copy(j % 4, j % 3).wait()
                    dst_copy(i, j % 4).wait()
                    scat_copy(j % 4, j % 3).start(add=True)

                @pl.when(i + 3 < cnt)
                def _():
                    src_copy(i + 3, (j + 3) % 4).start()

                @pl.when(i + 2 < cnt)
                def _():
                    dst_copy(i + 2, (j + 2) % 4).start()

        plsc.subcore_barrier()
        pltpu.sync_copy(
            agg_sh.at[pl.ds(s * nr, nr)], out_hbm.at[c, pl.ds(s * nr, nr)]
        )
        if tail:
            @pl.when(s == _NUM_SUBCORES - 1)
            def _():
                pltpu.sync_copy(
                    agg_sh.at[pl.ds(nr * _NUM_SUBCORES, tail)],
                    out_hbm.at[c, pl.ds(nr * _NUM_SUBCORES, tail)],
                )

    return k(x, edge_index, zeros)


def _post(agg2, x, W_msg, W_self, b_row, att_w_row, batch_row, G):
    N, D = x.shape

    def body(agg_ref, x_ref, wm_ref, ws_ref, b_ref, aw_ref, bat_ref, out_ref):
        aggm = jnp.dot(
            agg_ref[0] + agg_ref[1], wm_ref[...],
            preferred_element_type=jnp.float32,
        )
        x_self = (
            jnp.dot(x_ref[...], ws_ref[...], preferred_element_type=jnp.float32)
            + b_ref[...]
        )
        h = jnp.maximum(aggm + x_self, 0.0)
        logits = lax.dot_general(
            aw_ref[...], h, (((1,), (1,)), ((), ())),
            preferred_element_type=jnp.float32,
        )
        bat = bat_ref[...]
        gid = lax.broadcasted_iota(jnp.int32, (G, N), 0)
        mask = gid == bat
        big_neg = jnp.float32(-1e30)
        seg_max = jnp.max(jnp.where(mask, logits, big_neg), axis=1, keepdims=True)
        maxn = jnp.sum(jnp.where(mask, seg_max, 0.0), axis=0, keepdims=True)
        w = jnp.exp(logits - maxn)
        denom = jnp.sum(jnp.where(mask, w, 0.0), axis=1, keepdims=True)
        denn = jnp.sum(jnp.where(mask, denom, 0.0), axis=0, keepdims=True)
        wn = w / denn
        out_ref[...] = jnp.dot(
            jnp.where(mask, wn, 0.0), h, preferred_element_type=jnp.float32
        )

    return pl.pallas_call(
        body, out_shape=jax.ShapeDtypeStruct((G, D), jnp.float32)
    )(agg2, x, W_msg, W_self, b_row, att_w_row, batch_row)


def kernel(x, edge_index, batch, W_msg, W_self, b, att_w, att_b):
    N, D = x.shape
    G = 64
    zeros = jnp.zeros((640, D), jnp.float32)
    agg2 = _edge_agg_sc(x, edge_index, zeros)
    out = _post(agg2, x, W_msg, W_self, b.reshape(1, D), att_w.reshape(1, D),
                batch.reshape(1, N), G)
    return out
